# baseline (speedup 1.0000x reference)
import jax
import jax.numpy as jnp
from jax.experimental import pallas as pl
from jax.experimental.pallas import tpu as pltpu

SCALING = 32.0 / 16.0
N_CHUNKS = 8


def _moe_lora_kernel(x_ref, wb_ref, b_ref, wr_ref, a_ref, bl_ref, sel_ref,
                     o_ref, wvm_ref, sem_ref):
    pid = pl.program_id(0)
    d_out = wvm_ref.shape[0]
    csize = d_out // N_CHUNKS

    def _w_copy(c):
        return pltpu.make_async_copy(
            wb_ref.at[pl.ds(c * csize, csize), :],
            wvm_ref.at[pl.ds(c * csize, csize), :],
            sem_ref.at[c])

    @pl.when(pid == 0)
    def _start_copies():
        for c in range(N_CHUNKS):
            _w_copy(c).start()

    x = x_ref[...]

    logits = jax.lax.dot_general(x, wr_ref[...], (((1,), (1,)), ((), ())),
                                 preferred_element_type=jnp.float32)

    mid = jax.lax.dot_general(x, a_ref[...], (((1,), (0,)), ((), ())),
                              preferred_element_type=jnp.float32)

    m1 = jnp.max(logits, axis=1, keepdims=True)
    mask1 = logits == m1
    l_rest = jnp.where(mask1, -jnp.inf, logits)
    m2 = jnp.max(l_rest, axis=1, keepdims=True)
    mask2 = l_rest == m2
    w1 = 1.0 / (1.0 + jnp.exp(m2 - m1))
    w = jnp.where(mask1, w1, 0.0) + jnp.where(mask2, 1.0 - w1, 0.0)
    w = w * SCALING

    wrep = jax.lax.dot_general(w, sel_ref[...], (((1,), (0,)), ((), ())),
                               preferred_element_type=jnp.float32)
    lw = mid * wrep

    for c in range(N_CHUNKS):
        @pl.when(pid == 0)
        def _wait_chunk(c=c):
            _w_copy(c).wait()

        w_chunk = wvm_ref[pl.ds(c * csize, csize), :]
        base_c = jax.lax.dot_general(x, w_chunk, (((1,), (1,)), ((), ())),
                                     preferred_element_type=jnp.float32)
        lora_c = jax.lax.dot_general(lw, bl_ref[:, pl.ds(c * csize, csize)],
                                     (((1,), (0,)), ((), ())),
                                     preferred_element_type=jnp.float32)
        o_ref[:, pl.ds(c * csize, csize)] = (
            base_c + b_ref[:, pl.ds(c * csize, csize)] + lora_c)


def kernel(x, W_base, b_base, W_router, lora_A, lora_B):
    b, s, d_in = x.shape
    d_out = W_base.shape[0]
    e, _, r = lora_A.shape
    n = b * s
    n_tile = 512

    x2 = x.reshape(n, d_in)
    a_flat = lora_A.transpose(1, 0, 2).reshape(d_in, e * r)
    b_flat = lora_B.reshape(e * r, d_out)
    sel = jnp.kron(jnp.eye(e, dtype=x.dtype), jnp.ones((1, r), dtype=x.dtype))
    bias2 = b_base.reshape(1, d_out)

    out = pl.pallas_call(
        _moe_lora_kernel,
        grid=(n // n_tile,),
        in_specs=[
            pl.BlockSpec((n_tile, d_in), lambda i: (i, 0)),
            pl.BlockSpec(memory_space=pltpu.MemorySpace.HBM),
            pl.BlockSpec((1, d_out), lambda i: (0, 0)),
            pl.BlockSpec((e, d_in), lambda i: (0, 0)),
            pl.BlockSpec((d_in, e * r), lambda i: (0, 0)),
            pl.BlockSpec((e * r, d_out), lambda i: (0, 0)),
            pl.BlockSpec((e, e * r), lambda i: (0, 0)),
        ],
        out_specs=pl.BlockSpec((n_tile, d_out), lambda i: (i, 0)),
        out_shape=jax.ShapeDtypeStruct((n, d_out), x.dtype),
        scratch_shapes=[
            pltpu.VMEM((d_out, d_in), jnp.float32),
            pltpu.SemaphoreType.DMA((N_CHUNKS,)),
        ],
    )(x2, W_base, bias2, W_router, a_flat, b_flat, sel)

    return out.reshape(b, s, d_out)


# speedup vs baseline: 1.5381x; 1.5381x over previous
import jax
import jax.numpy as jnp
from jax.experimental import pallas as pl
from jax.experimental.pallas import tpu as pltpu

SCALING = 32.0 / 16.0
N_CHUNKS = 2


def _moe_lora_kernel(x_ref, wb_ref, b_ref, wr_ref, a_ref, bl_ref, sel_ref,
                     o_ref, wvm_ref, sem_ref):
    pid = pl.program_id(0)
    d_out = wvm_ref.shape[0]
    csize = d_out // N_CHUNKS

    def _w_copy(c):
        return pltpu.make_async_copy(
            wb_ref.at[pl.ds(c * csize, csize), :],
            wvm_ref.at[pl.ds(c * csize, csize), :],
            sem_ref.at[c])

    @pl.when(pid == 0)
    def _start_copies():
        for c in range(N_CHUNKS):
            _w_copy(c).start()

    x = x_ref[...]

    logits = jax.lax.dot_general(x, wr_ref[...], (((1,), (1,)), ((), ())),
                                 preferred_element_type=jnp.float32)

    mid = jax.lax.dot_general(x, a_ref[...], (((1,), (0,)), ((), ())),
                              preferred_element_type=jnp.float32)

    m1 = jnp.max(logits, axis=1, keepdims=True)
    mask1 = logits == m1
    l_rest = jnp.where(mask1, -jnp.inf, logits)
    m2 = jnp.max(l_rest, axis=1, keepdims=True)
    mask2 = l_rest == m2
    w1 = 1.0 / (1.0 + jnp.exp(m2 - m1))
    w = jnp.where(mask1, w1, 0.0) + jnp.where(mask2, 1.0 - w1, 0.0)
    w = w * SCALING

    wrep = jax.lax.dot_general(w, sel_ref[...], (((1,), (0,)), ((), ())),
                               preferred_element_type=jnp.float32)
    lw = mid * wrep

    for c in range(N_CHUNKS):
        @pl.when(pid == 0)
        def _wait_chunk(c=c):
            _w_copy(c).wait()

        w_chunk = wvm_ref[pl.ds(c * csize, csize), :]
        base_c = jax.lax.dot_general(x, w_chunk, (((1,), (1,)), ((), ())),
                                     preferred_element_type=jnp.float32)
        lora_c = jax.lax.dot_general(lw, bl_ref[:, pl.ds(c * csize, csize)],
                                     (((1,), (0,)), ((), ())),
                                     preferred_element_type=jnp.float32)
        o_ref[:, pl.ds(c * csize, csize)] = (
            base_c + b_ref[:, pl.ds(c * csize, csize)] + lora_c)


def kernel(x, W_base, b_base, W_router, lora_A, lora_B):
    b, s, d_in = x.shape
    d_out = W_base.shape[0]
    e, _, r = lora_A.shape
    n = b * s
    n_tile = 512

    x2 = x.reshape(n, d_in)
    a_flat = lora_A.transpose(1, 0, 2).reshape(d_in, e * r)
    b_flat = lora_B.reshape(e * r, d_out)
    sel = jnp.kron(jnp.eye(e, dtype=x.dtype), jnp.ones((1, r), dtype=x.dtype))
    bias2 = b_base.reshape(1, d_out)

    out = pl.pallas_call(
        _moe_lora_kernel,
        grid=(n // n_tile,),
        in_specs=[
            pl.BlockSpec((n_tile, d_in), lambda i: (i, 0)),
            pl.BlockSpec(memory_space=pltpu.MemorySpace.HBM),
            pl.BlockSpec((1, d_out), lambda i: (0, 0)),
            pl.BlockSpec((e, d_in), lambda i: (0, 0)),
            pl.BlockSpec((d_in, e * r), lambda i: (0, 0)),
            pl.BlockSpec((e * r, d_out), lambda i: (0, 0)),
            pl.BlockSpec((e, e * r), lambda i: (0, 0)),
        ],
        out_specs=pl.BlockSpec((n_tile, d_out), lambda i: (i, 0)),
        out_shape=jax.ShapeDtypeStruct((n, d_out), x.dtype),
        scratch_shapes=[
            pltpu.VMEM((d_out, d_in), jnp.float32),
            pltpu.SemaphoreType.DMA((N_CHUNKS,)),
        ],
    )(x2, W_base, bias2, W_router, a_flat, b_flat, sel)

    return out.reshape(b, s, d_out)
